# Initial kernel scaffold; baseline (speedup 1.0000x reference)
import functools

import jax
import jax.numpy as jnp
from jax import lax
from jax.experimental import pallas as pl
from jax.experimental.pallas import tpu as pltpu
from jax.experimental.pallas import tpu_sc as plsc

NUM_CORES = 2
NUM_SUBCORES = 16
NUM_WORKERS = NUM_CORES * NUM_SUBCORES
CHUNK = 128


def _make_gather(total_rows: int, dim: int):
    assert total_rows % (NUM_WORKERS * CHUNK) == 0
    rows_per_w = total_rows // NUM_WORKERS
    chunks_per_w = rows_per_w // CHUNK

    mesh = plsc.VectorSubcoreMesh(
        core_axis_name="c", subcore_axis_name="s",
        num_cores=NUM_CORES, num_subcores=NUM_SUBCORES)

    @functools.partial(
        pl.kernel,
        out_type=jax.ShapeDtypeStruct((total_rows, dim), jnp.float32),
        mesh=mesh,
        scratch_types=[
            pltpu.VMEM((chunks_per_w, CHUNK), jnp.int32),
            pltpu.VMEM((CHUNK, dim), jnp.float32),
            pltpu.SemaphoreType.DMA,
        ],
    )
    def gather_kernel(idx_hbm, table_hbm, out_hbm, idx_v, rows_v, gsem):
        wid = lax.axis_index("s") * NUM_CORES + lax.axis_index("c")
        chunk_base = wid * chunks_per_w
        row_base = wid * rows_per_w

        pltpu.sync_copy(idx_hbm.at[pl.ds(chunk_base, chunks_per_w)], idx_v)

        def body(g, _):
            pltpu.async_copy(table_hbm.at[idx_v.at[g]], rows_v, gsem).wait()
            pltpu.sync_copy(rows_v, out_hbm.at[pl.ds(row_base + g * CHUNK, CHUNK)])
            return 0

        lax.fori_loop(0, chunks_per_w, body, 0)

    return gather_kernel


def kernel(x, embedding):
    b, s = x.shape
    total = b * s
    idx2d = x.reshape(total // CHUNK, CHUNK).astype(jnp.int32)
    out = _make_gather(total, embedding.shape[1])(idx2d, embedding)
    return out.reshape(b, s, embedding.shape[1])


# speedup vs baseline: 1.6850x; 1.6850x over previous
import functools

import jax
import jax.numpy as jnp
from jax import lax
from jax.experimental import pallas as pl
from jax.experimental.pallas import tpu as pltpu
from jax.experimental.pallas import tpu_sc as plsc

NUM_CORES = 2
NUM_SUBCORES = 16
NUM_WORKERS = NUM_CORES * NUM_SUBCORES
CHUNK = 128


def _make_gather(total_rows: int, dim: int):
    assert total_rows % (NUM_WORKERS * CHUNK) == 0
    rows_per_w = total_rows // NUM_WORKERS
    chunks_per_w = rows_per_w // CHUNK

    mesh = plsc.VectorSubcoreMesh(
        core_axis_name="c", subcore_axis_name="s",
        num_cores=NUM_CORES, num_subcores=NUM_SUBCORES)

    @functools.partial(
        pl.kernel,
        out_type=jax.ShapeDtypeStruct((total_rows, dim), jnp.float32),
        mesh=mesh,
        compiler_params=pltpu.CompilerParams(use_tc_tiling_on_sc=False),
        scratch_types=[
            pltpu.VMEM((chunks_per_w, CHUNK), jnp.int32),
            pltpu.VMEM((CHUNK, dim), jnp.float32),
            pltpu.SemaphoreType.DMA,
        ],
    )
    def gather_kernel(idx_hbm, table_hbm, out_hbm, idx_v, rows_v, gsem):
        wid = lax.axis_index("s") * NUM_CORES + lax.axis_index("c")
        chunk_base = wid * chunks_per_w
        row_base = wid * rows_per_w

        pltpu.sync_copy(idx_hbm.at[pl.ds(chunk_base, chunks_per_w)], idx_v)

        def body(g, _):
            pltpu.async_copy(table_hbm.at[idx_v.at[g]], rows_v, gsem).wait()
            pltpu.sync_copy(rows_v, out_hbm.at[pl.ds(row_base + g * CHUNK, CHUNK)])
            return 0

        lax.fori_loop(0, chunks_per_w, body, 0)

    return gather_kernel


def kernel(x, embedding):
    b, s = x.shape
    total = b * s
    idx2d = x.reshape(total // CHUNK, CHUNK).astype(jnp.int32)
    out = _make_gather(total, embedding.shape[1])(idx2d, embedding)
    return out.reshape(b, s, embedding.shape[1])


# speedup vs baseline: 1.8759x; 1.1133x over previous
import functools

import jax
import jax.numpy as jnp
from jax import lax
from jax.experimental import pallas as pl
from jax.experimental.pallas import tpu as pltpu
from jax.experimental.pallas import tpu_sc as plsc

NUM_CORES = 2
NUM_SUBCORES = 16
NUM_WORKERS = NUM_CORES * NUM_SUBCORES
CHUNK = 128
GK = 4


def _make_gather(total_rows: int, dim: int):
    assert total_rows % (NUM_WORKERS * CHUNK * GK * 2) == 0
    rows_per_w = total_rows // NUM_WORKERS
    chunks_per_w = rows_per_w // CHUNK
    group_rows = GK * CHUNK
    num_pairs = chunks_per_w // (2 * GK)

    mesh = plsc.VectorSubcoreMesh(
        core_axis_name="c", subcore_axis_name="s",
        num_cores=NUM_CORES, num_subcores=NUM_SUBCORES)

    @functools.partial(
        pl.kernel,
        out_type=jax.ShapeDtypeStruct((total_rows, dim), jnp.float32),
        mesh=mesh,
        compiler_params=pltpu.CompilerParams(use_tc_tiling_on_sc=False),
        scratch_types=[
            pltpu.VMEM((chunks_per_w, CHUNK), jnp.int32),
            pltpu.VMEM((group_rows, dim), jnp.float32),
            pltpu.VMEM((group_rows, dim), jnp.float32),
            pltpu.SemaphoreType.DMA,
            pltpu.SemaphoreType.DMA,
            pltpu.SemaphoreType.DMA,
        ],
    )
    def gather_kernel(idx_hbm, table_hbm, out_hbm, idx_v, rows_a, rows_b,
                      gsem_a, gsem_b, osem):
        wid = lax.axis_index("s") * NUM_CORES + lax.axis_index("c")
        chunk_base = wid * chunks_per_w
        row_base = wid * rows_per_w

        pltpu.sync_copy(idx_hbm.at[pl.ds(chunk_base, chunks_per_w)], idx_v)

        def fire(group, buf, sem):
            for j in range(GK):
                pltpu.async_copy(
                    table_hbm.at[idx_v.at[group * GK + j]],
                    buf.at[pl.ds(j * CHUNK, CHUNK)], sem)

        def drain_gathers(buf, sem):
            pltpu.make_async_copy(
                out_hbm.at[pl.ds(0, group_rows)], buf, sem).wait()

        def scatter(group, buf):
            pltpu.async_copy(
                buf, out_hbm.at[pl.ds(row_base + group * group_rows,
                                      group_rows)], osem).wait()

        fire(0, rows_a, gsem_a)

        def body(q, _):
            a = 2 * q
            fire(a + 1, rows_b, gsem_b)
            drain_gathers(rows_a, gsem_a)
            scatter(a, rows_a)

            @pl.when(q < num_pairs - 1)
            def _():
                fire(a + 2, rows_a, gsem_a)

            drain_gathers(rows_b, gsem_b)
            scatter(a + 1, rows_b)
            return 0

        lax.fori_loop(0, num_pairs, body, 0)

    return gather_kernel


def kernel(x, embedding):
    b, s = x.shape
    total = b * s
    idx2d = x.reshape(total // CHUNK, CHUNK).astype(jnp.int32)
    out = _make_gather(total, embedding.shape[1])(idx2d, embedding)
    return out.reshape(b, s, embedding.shape[1])
